# baseline (speedup 1.0000x reference)
import functools

import jax
import jax.numpy as jnp
from jax.experimental import pallas as pl
from jax.experimental.pallas import tpu as pltpu


def _se_kernel_nchw(x_ref, w1_ref, w2_ref, o_ref, *, inv_hw):
    x = x_ref[...]
    pooled = jnp.sum(x, axis=2, dtype=jnp.float32) * inv_hw
    h = jax.lax.dot_general(pooled, w1_ref[...],
                            (((1,), (1,)), ((), ())),
                            preferred_element_type=jnp.float32)
    h = jnp.maximum(h, 0.2 * h)
    y = jnp.tanh(jax.lax.dot_general(h, w2_ref[...],
                                     (((1,), (1,)), ((), ())),
                                     preferred_element_type=jnp.float32))
    o_ref[...] = x * y[:, :, None].astype(o_ref.dtype)


def _se_kernel(x_ref, w1_ref, w2_ref, o_ref, *, inv_hw):
    x = x_ref[...]
    pooled = jnp.sum(x, axis=1, dtype=jnp.float32) * inv_hw
    h = jax.lax.dot_general(pooled, w1_ref[...],
                            (((1,), (1,)), ((), ())),
                            preferred_element_type=jnp.float32)
    h = jnp.maximum(h, 0.2 * h)
    y = jnp.tanh(jax.lax.dot_general(h, w2_ref[...],
                                     (((1,), (1,)), ((), ())),
                                     preferred_element_type=jnp.float32))
    o_ref[...] = x * y[:, None, :].astype(o_ref.dtype)


def _se_chunk(x, w1, w2):
    B, C, H, W = x.shape
    HW = H * W
    Cr = w1.shape[0]

    bytes_per_image = C * HW * x.dtype.itemsize
    tb_cap = max(1, (13 << 20) // bytes_per_image)
    tb = 1
    for cand in range(min(B, tb_cap), 0, -1):
        if B % cand == 0:
            tb = cand
            break

    x_t = x.reshape(B, C, HW).transpose(0, 2, 1)
    block = (tb, HW, C)
    block_bytes = tb * bytes_per_image
    vmem_limit = int(min(5 * block_bytes + (4 << 20), 56 << 20))

    out = pl.pallas_call(
        functools.partial(_se_kernel, inv_hw=1.0 / HW),
        out_shape=jax.ShapeDtypeStruct((B, HW, C), x.dtype),
        grid=(B // tb,),
        in_specs=[
            pl.BlockSpec(block, lambda b: (b, 0, 0)),
            pl.BlockSpec((Cr, C), lambda b: (0, 0)),
            pl.BlockSpec((C, Cr), lambda b: (0, 0)),
        ],
        out_specs=pl.BlockSpec(block, lambda b: (b, 0, 0)),
        compiler_params=pltpu.CompilerParams(
            dimension_semantics=("parallel",),
            vmem_limit_bytes=vmem_limit,
        ),
        cost_estimate=pl.CostEstimate(
            flops=2 * B * C * HW + 4 * B * C * Cr,
            transcendentals=B * C,
            bytes_accessed=2 * B * C * HW * x.dtype.itemsize,
        ),
    )(x_t, w1, w2)
    return out.transpose(0, 2, 1).reshape(B, C, H, W)


def _se_chunk_nchw(x, w1, w2):
    B, C, H, W = x.shape
    HW = H * W
    Cr = w1.shape[0]

    bytes_per_image = C * (-(-HW // 128) * 128) * x.dtype.itemsize
    tb_cap = max(1, (10 << 20) // bytes_per_image)
    tb = 1
    for cand in range(min(B, tb_cap), 0, -1):
        if B % cand == 0:
            tb = cand
            break

    x_k = x.reshape(B, C, HW)
    block = (tb, C, HW)
    block_bytes = tb * bytes_per_image
    vmem_limit = int(min(5 * block_bytes + (4 << 20), 56 << 20))

    out = pl.pallas_call(
        functools.partial(_se_kernel_nchw, inv_hw=1.0 / HW),
        out_shape=jax.ShapeDtypeStruct((B, C, HW), x.dtype),
        grid=(B // tb,),
        in_specs=[
            pl.BlockSpec(block, lambda b: (b, 0, 0)),
            pl.BlockSpec((Cr, C), lambda b: (0, 0)),
            pl.BlockSpec((C, Cr), lambda b: (0, 0)),
        ],
        out_specs=pl.BlockSpec(block, lambda b: (b, 0, 0)),
        compiler_params=pltpu.CompilerParams(
            dimension_semantics=("parallel",),
            vmem_limit_bytes=vmem_limit,
        ),
        cost_estimate=pl.CostEstimate(
            flops=2 * B * C * HW + 4 * B * C * Cr,
            transcendentals=B * C,
            bytes_accessed=2 * B * C * HW * x.dtype.itemsize,
        ),
    )(x_k, w1, w2)
    return out.reshape(B, C, H, W)


def kernel(x, w1, w2):
    B = x.shape[0]
    split = (B // 4) if B % 4 == 0 else 0
    if split == 0:
        return _se_chunk(x, w1, w2)
    out_a = _se_chunk_nchw(x[:split], w1, w2)
    out_b = _se_chunk(x[split:], w1, w2)
    return jnp.concatenate([out_a, out_b], axis=0)


# speedup vs baseline: 1.6744x; 1.6744x over previous
import functools

import jax
import jax.numpy as jnp
from jax.experimental import pallas as pl
from jax.experimental.pallas import tpu as pltpu


def _se_kernel_nchw(x_ref, w1_ref, w2_ref, o_ref, *, inv_hw):
    x = x_ref[...]
    pooled = jnp.sum(x, axis=2, dtype=jnp.float32) * inv_hw
    h = jax.lax.dot_general(pooled, w1_ref[...],
                            (((1,), (1,)), ((), ())),
                            preferred_element_type=jnp.float32)
    h = jnp.maximum(h, 0.2 * h)
    y = jnp.tanh(jax.lax.dot_general(h, w2_ref[...],
                                     (((1,), (1,)), ((), ())),
                                     preferred_element_type=jnp.float32))
    o_ref[...] = x * y[:, :, None].astype(o_ref.dtype)


def _se_kernel(x_ref, w1_ref, w2_ref, o_ref, *, inv_hw):
    x = x_ref[...]
    pooled = jnp.sum(x, axis=1, dtype=jnp.float32) * inv_hw
    h = jax.lax.dot_general(pooled, w1_ref[...],
                            (((1,), (1,)), ((), ())),
                            preferred_element_type=jnp.float32)
    h = jnp.maximum(h, 0.2 * h)
    y = jnp.tanh(jax.lax.dot_general(h, w2_ref[...],
                                     (((1,), (1,)), ((), ())),
                                     preferred_element_type=jnp.float32))
    o_ref[...] = x * y[:, None, :].astype(o_ref.dtype)


def _se_chunk(x, w1, w2):
    B, C, H, W = x.shape
    HW = H * W
    Cr = w1.shape[0]

    bytes_per_image = C * HW * x.dtype.itemsize
    tb_cap = max(1, (13 << 20) // bytes_per_image)
    tb = 1
    for cand in range(min(B, tb_cap), 0, -1):
        if B % cand == 0:
            tb = cand
            break

    x_t = x.reshape(B, C, HW).transpose(0, 2, 1)
    block = (tb, HW, C)
    block_bytes = tb * bytes_per_image
    vmem_limit = int(min(5 * block_bytes + (4 << 20), 56 << 20))

    out = pl.pallas_call(
        functools.partial(_se_kernel, inv_hw=1.0 / HW),
        out_shape=jax.ShapeDtypeStruct((B, HW, C), x.dtype),
        grid=(B // tb,),
        in_specs=[
            pl.BlockSpec(block, lambda b: (b, 0, 0)),
            pl.BlockSpec((Cr, C), lambda b: (0, 0)),
            pl.BlockSpec((C, Cr), lambda b: (0, 0)),
        ],
        out_specs=pl.BlockSpec(block, lambda b: (b, 0, 0)),
        compiler_params=pltpu.CompilerParams(
            dimension_semantics=("parallel",),
            vmem_limit_bytes=vmem_limit,
        ),
        cost_estimate=pl.CostEstimate(
            flops=2 * B * C * HW + 4 * B * C * Cr,
            transcendentals=B * C,
            bytes_accessed=2 * B * C * HW * x.dtype.itemsize,
        ),
    )(x_t, w1, w2)
    return out.transpose(0, 2, 1).reshape(B, C, H, W)


def _se_chunk_nchw(x, w1, w2):
    B, C, H, W = x.shape
    HW = H * W
    Cr = w1.shape[0]

    bytes_per_image = C * (-(-HW // 128) * 128) * x.dtype.itemsize
    tb_cap = max(1, (10 << 20) // bytes_per_image)
    tb = 1
    for cand in range(min(B, tb_cap), 0, -1):
        if B % cand == 0:
            tb = cand
            break

    x_k = x.reshape(B, C, HW)
    block = (tb, C, HW)
    block_bytes = tb * bytes_per_image
    vmem_limit = int(min(5 * block_bytes + (4 << 20), 56 << 20))

    out = pl.pallas_call(
        functools.partial(_se_kernel_nchw, inv_hw=1.0 / HW),
        out_shape=jax.ShapeDtypeStruct((B, C, HW), x.dtype),
        grid=(B // tb,),
        in_specs=[
            pl.BlockSpec(block, lambda b: (b, 0, 0)),
            pl.BlockSpec((Cr, C), lambda b: (0, 0)),
            pl.BlockSpec((C, Cr), lambda b: (0, 0)),
        ],
        out_specs=pl.BlockSpec(block, lambda b: (b, 0, 0)),
        compiler_params=pltpu.CompilerParams(
            dimension_semantics=("parallel",),
            vmem_limit_bytes=vmem_limit,
        ),
        cost_estimate=pl.CostEstimate(
            flops=2 * B * C * HW + 4 * B * C * Cr,
            transcendentals=B * C,
            bytes_accessed=2 * B * C * HW * x.dtype.itemsize,
        ),
    )(x_k, w1, w2)
    return out.reshape(B, C, H, W)


def kernel(x, w1, w2):
    B = x.shape[0]
    return _se_chunk(x, w1, w2)


# speedup vs baseline: 1.6788x; 1.0026x over previous
import functools

import jax
import jax.numpy as jnp
from jax.experimental import pallas as pl
from jax.experimental.pallas import tpu as pltpu


def _se_kernel(x_ref, w1_ref, w2_ref, o_ref, *, inv_hw):
    x = x_ref[...]
    pooled = jnp.sum(x, axis=1, dtype=jnp.float32) * inv_hw
    h = jax.lax.dot_general(pooled, w1_ref[...],
                            (((1,), (1,)), ((), ())),
                            preferred_element_type=jnp.float32)
    h = jnp.maximum(h, 0.2 * h)
    y = jnp.tanh(jax.lax.dot_general(h, w2_ref[...],
                                     (((1,), (1,)), ((), ())),
                                     preferred_element_type=jnp.float32))
    o_ref[...] = x * y[:, None, :].astype(o_ref.dtype)


def kernel(x, w1, w2):
    B, C, H, W = x.shape
    HW = H * W
    Cr = w1.shape[0]

    bytes_per_image = C * HW * x.dtype.itemsize
    tb_cap = max(1, (13 << 20) // bytes_per_image)
    tb = 1
    for cand in range(min(B, tb_cap), 0, -1):
        if B % cand == 0:
            tb = cand
            break

    x_t = x.reshape(B, C, HW).transpose(0, 2, 1)
    block = (tb, HW, C)
    block_bytes = tb * bytes_per_image
    vmem_limit = int(min(5 * block_bytes + (4 << 20), 56 << 20))

    out = pl.pallas_call(
        functools.partial(_se_kernel, inv_hw=1.0 / HW),
        out_shape=jax.ShapeDtypeStruct((B, HW, C), x.dtype),
        grid=(B // tb,),
        in_specs=[
            pl.BlockSpec(block, lambda b: (b, 0, 0)),
            pl.BlockSpec((Cr, C), lambda b: (0, 0)),
            pl.BlockSpec((C, Cr), lambda b: (0, 0)),
        ],
        out_specs=pl.BlockSpec(block, lambda b: (b, 0, 0)),
        compiler_params=pltpu.CompilerParams(
            dimension_semantics=("parallel",),
            vmem_limit_bytes=vmem_limit,
        ),
        cost_estimate=pl.CostEstimate(
            flops=2 * B * C * HW + 4 * B * C * Cr,
            transcendentals=B * C,
            bytes_accessed=2 * B * C * HW * x.dtype.itemsize,
        ),
    )(x_t, w1, w2)
    return out.transpose(0, 2, 1).reshape(B, C, H, W)
